# Initial kernel scaffold; baseline (speedup 1.0000x reference)
import jax
import jax.numpy as jnp
from jax.experimental import pallas as pl


def kernel(xs, pos_enc, W1, a_src1, a_dst1, b1, W2, a_src2, a_dst2, b2, W3, a_src3, a_dst3, b3):
    raise NotImplementedError("write your pallas kernel here")


# speedup vs baseline: 2717.6971x; 2717.6971x over previous
import jax
import jax.numpy as jnp
from jax import lax
from jax.experimental import pallas as pl

_N = 128
_HEADS = 4
_HID = 32
_OUT_DIM = 2


def _gat_block(h, asrc_t, adst_t, bias, outd):
    ht = h.T
    als = jnp.dot(asrc_t, ht)
    ald = jnp.dot(adst_t, ht)
    e = als[:, :, None] + ald[:, None, :]
    e = jnp.where(e > 0, e, 0.2 * e)
    m = jnp.max(e, axis=1, keepdims=True)
    ex = jnp.exp(e - m)
    den = jnp.sum(ex, axis=1, keepdims=True)
    alpha = ex / (den + 1e-16)
    outs = []
    for hd in range(_HEADS):
        h_h = lax.slice(h, (0, hd * outd), (_N, (hd + 1) * outd))
        o_h = lax.dot_general(alpha[hd], h_h, (((0,), (0,)), ((), ())))
        outs.append(o_h)
    return jnp.concatenate(outs, axis=1) + bias


def _block_kernel(x_ref, w1_ref, as1_ref, ad1_ref, b1_ref,
                  w2_ref, as2_ref, ad2_ref, b2_ref,
                  w3_ref, as3_ref, ad3_ref, b3_ref, out_ref):
    x = x_ref[0]
    h1 = jnp.dot(x, w1_ref[...])
    o1 = _gat_block(h1, as1_ref[...], ad1_ref[...], b1_ref[...], _HID)
    h2 = jnp.dot(o1, w2_ref[...])
    o2 = _gat_block(h2, as2_ref[...], ad2_ref[...], b2_ref[...], _HID)
    h3 = jnp.dot(o2, w3_ref[...])
    o3 = _gat_block(h3, as3_ref[...], ad3_ref[...], b3_ref[...], _OUT_DIM)
    out_ref[0, 0, :] = jnp.mean(o3, axis=0)


def _attn_mat(a):
    heads, outd = a.shape
    return (jnp.eye(heads, dtype=a.dtype)[:, :, None] * a[None, :, :]).reshape(
        heads, heads * outd)


def kernel(xs, pos_enc, W1, a_src1, a_dst1, b1, W2, a_src2, a_dst2, b2,
           W3, a_src3, a_dst3, b3):
    bs, nr, nc = xs.shape
    enc = pos_enc.shape[-1]
    nblocks = bs * nr
    pe = jnp.broadcast_to(pos_enc[:, None, :, :], (bs, nr, nc, enc))
    x = jnp.concatenate([xs[..., None], pe], axis=-1).reshape(nblocks, nc, 1 + enc)

    def whole(shape):
        return pl.BlockSpec(shape, lambda i: tuple(0 for _ in shape))

    hh = _HEADS * _HID
    ho = _HEADS * _OUT_DIM
    out = pl.pallas_call(
        _block_kernel,
        grid=(nblocks,),
        in_specs=[
            pl.BlockSpec((1, nc, 1 + enc), lambda i: (i, 0, 0)),
            whole((1 + enc, hh)),
            whole((_HEADS, hh)), whole((_HEADS, hh)), whole((1, hh)),
            whole((hh, hh)),
            whole((_HEADS, hh)), whole((_HEADS, hh)), whole((1, hh)),
            whole((hh, ho)),
            whole((_HEADS, ho)), whole((_HEADS, ho)), whole((1, ho)),
        ],
        out_specs=pl.BlockSpec((1, 1, _HEADS * _OUT_DIM), lambda i: (i, 0, 0)),
        out_shape=jax.ShapeDtypeStruct((nblocks, 1, _HEADS * _OUT_DIM),
                                       jnp.float32),
    )(
        x,
        W1, _attn_mat(a_src1), _attn_mat(a_dst1), b1.reshape(1, -1),
        W2, _attn_mat(a_src2), _attn_mat(a_dst2), b2.reshape(1, -1),
        W3, _attn_mat(a_src3), _attn_mat(a_dst3), b3.reshape(1, -1),
    )
    return out.reshape(bs, nr, _HEADS * _OUT_DIM)


# speedup vs baseline: 3266.0049x; 1.2018x over previous
import jax
import jax.numpy as jnp
from jax import lax
from jax.experimental import pallas as pl
from jax.experimental.pallas import tpu as pltpu

_N = 128
_HEADS = 4
_HID = 32
_OUT_DIM = 2
_NB = 4


def _gat_block(h, als, ald, outd):
    als_t = als.T
    ald_t = ald.T
    e = als_t[:, :, None] + ald_t[:, None, :]
    e = jnp.where(e > 0, e, 0.2 * e)
    m = jnp.max(e, axis=1, keepdims=True)
    ex = jnp.exp(e - m)
    den = jnp.sum(ex, axis=1, keepdims=True)
    alpha = ex / (den + 1e-16)
    outs = []
    for hd in range(_HEADS):
        h_h = lax.slice(h, (0, hd * outd), (_N, (hd + 1) * outd))
        o_h = lax.dot_general(alpha[hd], h_h, (((0,), (0,)), ((), ())))
        outs.append(o_h)
    return jnp.concatenate(outs, axis=1)


def _layer(hf, asrc_m, adst_m, bias, outd):
    als_f = jnp.dot(hf, asrc_m)
    ald_f = jnp.dot(hf, adst_m)
    outs = []
    for b in range(_NB):
        sl = slice(b * _N, (b + 1) * _N)
        outs.append(_gat_block(hf[sl], als_f[sl], ald_f[sl], outd))
    return jnp.concatenate(outs, axis=0) + bias


def _block_kernel(x_ref, w1_ref, as1_ref, ad1_ref, b1_ref,
                  w2_ref, as2_ref, ad2_ref, b2_ref,
                  w3_ref, as3_ref, ad3_ref, b3_ref, out_ref):
    x = x_ref[...].reshape(_NB * _N, _N)
    h1 = jnp.dot(x, w1_ref[...])
    o1 = _layer(h1, as1_ref[...], ad1_ref[...], b1_ref[...], _HID)
    h2 = jnp.dot(o1, w2_ref[...])
    o2 = _layer(h2, as2_ref[...], ad2_ref[...], b2_ref[...], _HID)
    h3 = jnp.dot(o2, w3_ref[...])
    o3 = _layer(h3, as3_ref[...], ad3_ref[...], b3_ref[...], _OUT_DIM)
    for b in range(_NB):
        blk = lax.slice(o3, (b * _N, 0), ((b + 1) * _N, _HEADS * _OUT_DIM))
        out_ref[b, 0, :] = jnp.mean(blk, axis=0)


def _attn_mat(a):
    heads, outd = a.shape
    return (jnp.eye(heads, dtype=a.dtype)[:, :, None] * a[None, :, :]).reshape(
        heads, heads * outd).T


def kernel(xs, pos_enc, W1, a_src1, a_dst1, b1, W2, a_src2, a_dst2, b2,
           W3, a_src3, a_dst3, b3):
    bs, nr, nc = xs.shape
    enc = pos_enc.shape[-1]
    nblocks = bs * nr
    pe = jnp.broadcast_to(pos_enc[:, None, :, :], (bs, nr, nc, enc))
    x = jnp.concatenate([xs[..., None], pe], axis=-1).reshape(nblocks, nc, 1 + enc)

    def whole(shape):
        return pl.BlockSpec(shape, lambda i: tuple(0 for _ in shape))

    hh = _HEADS * _HID
    ho = _HEADS * _OUT_DIM
    out = pl.pallas_call(
        _block_kernel,
        grid=(nblocks // _NB,),
        in_specs=[
            pl.BlockSpec((_NB, nc, 1 + enc), lambda i: (i, 0, 0)),
            whole((1 + enc, hh)),
            whole((hh, _HEADS)), whole((hh, _HEADS)), whole((1, hh)),
            whole((hh, hh)),
            whole((hh, _HEADS)), whole((hh, _HEADS)), whole((1, hh)),
            whole((hh, ho)),
            whole((ho, _HEADS)), whole((ho, _HEADS)), whole((1, ho)),
        ],
        out_specs=pl.BlockSpec((_NB, 1, ho), lambda i: (i, 0, 0)),
        out_shape=jax.ShapeDtypeStruct((nblocks, 1, ho), jnp.float32),
        compiler_params=pltpu.CompilerParams(
            dimension_semantics=("parallel",)),
    )(
        x,
        W1, _attn_mat(a_src1), _attn_mat(a_dst1), b1.reshape(1, -1),
        W2, _attn_mat(a_src2), _attn_mat(a_dst2), b2.reshape(1, -1),
        W3, _attn_mat(a_src3), _attn_mat(a_dst3), b3.reshape(1, -1),
    )
    return out.reshape(bs, nr, _HEADS * _OUT_DIM)


# speedup vs baseline: 3474.5263x; 1.0638x over previous
import jax
import jax.numpy as jnp
from jax import lax
from jax.experimental import pallas as pl
from jax.experimental.pallas import tpu as pltpu

_N = 128
_HEADS = 4
_HID = 32
_OUT_DIM = 2
_NB = 4


def _gat_block(h, als, ald, outd):
    ald_t = ald.T
    outs = []
    for hd in range(_HEADS):
        e = (lax.slice(als, (0, hd), (_N, hd + 1)) +
             lax.slice(ald_t, (hd, 0), (hd + 1, _N)))
        e = jnp.where(e > 0, e, 0.2 * e)
        m = jnp.max(e, axis=0, keepdims=True)
        ex = jnp.exp(e - m)
        den = jnp.sum(ex, axis=0, keepdims=True)
        alpha = ex / (den + 1e-16)
        h_h = lax.slice(h, (0, hd * outd), (_N, (hd + 1) * outd))
        outs.append(lax.dot_general(alpha, h_h, (((0,), (0,)), ((), ()))))
    return jnp.concatenate(outs, axis=1)


def _layer(hf, asrc_m, adst_m, bias, outd):
    als_f = jnp.dot(hf, asrc_m)
    ald_f = jnp.dot(hf, adst_m)
    outs = []
    for b in range(_NB):
        sl = slice(b * _N, (b + 1) * _N)
        outs.append(_gat_block(hf[sl], als_f[sl], ald_f[sl], outd))
    return jnp.concatenate(outs, axis=0) + bias


def _block_kernel(x_ref, w1_ref, as1_ref, ad1_ref, b1_ref,
                  w2_ref, as2_ref, ad2_ref, b2_ref,
                  w3_ref, as3_ref, ad3_ref, b3_ref, out_ref):
    x = x_ref[...].reshape(_NB * _N, _N)
    h1 = jnp.dot(x, w1_ref[...])
    o1 = _layer(h1, as1_ref[...], ad1_ref[...], b1_ref[...], _HID)
    h2 = jnp.dot(o1, w2_ref[...])
    o2 = _layer(h2, as2_ref[...], ad2_ref[...], b2_ref[...], _HID)
    h3 = jnp.dot(o2, w3_ref[...])
    o3 = _layer(h3, as3_ref[...], ad3_ref[...], b3_ref[...], _OUT_DIM)
    for b in range(_NB):
        blk = lax.slice(o3, (b * _N, 0), ((b + 1) * _N, _HEADS * _OUT_DIM))
        out_ref[b, 0, :] = jnp.mean(blk, axis=0)


def _attn_mat(a):
    heads, outd = a.shape
    return (jnp.eye(heads, dtype=a.dtype)[:, :, None] * a[None, :, :]).reshape(
        heads, heads * outd).T


def kernel(xs, pos_enc, W1, a_src1, a_dst1, b1, W2, a_src2, a_dst2, b2,
           W3, a_src3, a_dst3, b3):
    bs, nr, nc = xs.shape
    enc = pos_enc.shape[-1]
    nblocks = bs * nr
    pe = jnp.broadcast_to(pos_enc[:, None, :, :], (bs, nr, nc, enc))
    x = jnp.concatenate([xs[..., None], pe], axis=-1).reshape(nblocks, nc, 1 + enc)

    def whole(shape):
        return pl.BlockSpec(shape, lambda i: tuple(0 for _ in shape))

    hh = _HEADS * _HID
    ho = _HEADS * _OUT_DIM
    out = pl.pallas_call(
        _block_kernel,
        grid=(nblocks // _NB,),
        in_specs=[
            pl.BlockSpec((_NB, nc, 1 + enc), lambda i: (i, 0, 0)),
            whole((1 + enc, hh)),
            whole((hh, _HEADS)), whole((hh, _HEADS)), whole((1, hh)),
            whole((hh, hh)),
            whole((hh, _HEADS)), whole((hh, _HEADS)), whole((1, hh)),
            whole((hh, ho)),
            whole((ho, _HEADS)), whole((ho, _HEADS)), whole((1, ho)),
        ],
        out_specs=pl.BlockSpec((_NB, 1, ho), lambda i: (i, 0, 0)),
        out_shape=jax.ShapeDtypeStruct((nblocks, 1, ho), jnp.float32),
        compiler_params=pltpu.CompilerParams(
            dimension_semantics=("parallel",)),
    )(
        x,
        W1, _attn_mat(a_src1), _attn_mat(a_dst1), b1.reshape(1, -1),
        W2, _attn_mat(a_src2), _attn_mat(a_dst2), b2.reshape(1, -1),
        W3, _attn_mat(a_src3), _attn_mat(a_dst3), b3.reshape(1, -1),
    )
    return out.reshape(bs, nr, _HEADS * _OUT_DIM)


# speedup vs baseline: 4045.3163x; 1.1643x over previous
import jax
import jax.numpy as jnp
from jax import lax
from jax.experimental import pallas as pl
from jax.experimental.pallas import tpu as pltpu

_N = 128
_HEADS = 4
_HID = 32
_OUT_DIM = 2
_NB = 4


def _layer(hf, asrc_m, adst_m, mask4, bias, outd):
    als_f = jnp.dot(hf, asrc_m)
    ald_f = jnp.dot(hf, adst_m)
    lhs_f = jnp.concatenate([als_f, jnp.ones_like(als_f)], axis=1)
    outs = []
    for b in range(_NB):
        r0 = b * _N
        ald_t = lax.slice(ald_f, (r0, 0), (r0 + _N, _HEADS)).T
        ald_tile = jnp.concatenate([ald_t] * _HEADS, axis=1)
        rhs = jnp.concatenate([mask4, ald_tile * mask4], axis=0)
        e = jnp.dot(lax.slice(lhs_f, (r0, 0), (r0 + _N, 2 * _HEADS)), rhs)
        e = jnp.where(e > 0, e, 0.2 * e)
        m = jnp.max(e, axis=0, keepdims=True)
        ex = jnp.exp(e - m)
        den = jnp.sum(ex, axis=0, keepdims=True)
        alpha = ex / (den + 1e-16)
        oh = []
        for hd in range(_HEADS):
            a_h = lax.slice(alpha, (0, hd * _N), (_N, (hd + 1) * _N))
            h_h = lax.slice(hf, (r0, hd * outd), (r0 + _N, (hd + 1) * outd))
            oh.append(lax.dot_general(a_h, h_h, (((0,), (0,)), ((), ()))))
        outs.append(jnp.concatenate(oh, axis=1))
    return jnp.concatenate(outs, axis=0) + bias


def _block_kernel(x_ref, mask_ref, w1_ref, as1_ref, ad1_ref, b1_ref,
                  w2_ref, as2_ref, ad2_ref, b2_ref,
                  w3_ref, as3_ref, ad3_ref, b3_ref, out_ref):
    mask4 = mask_ref[...]
    x = x_ref[...].reshape(_NB * _N, _N)
    h1 = jnp.dot(x, w1_ref[...])
    o1 = _layer(h1, as1_ref[...], ad1_ref[...], mask4, b1_ref[...], _HID)
    h2 = jnp.dot(o1, w2_ref[...])
    o2 = _layer(h2, as2_ref[...], ad2_ref[...], mask4, b2_ref[...], _HID)
    h3 = jnp.dot(o2, w3_ref[...])
    o3 = _layer(h3, as3_ref[...], ad3_ref[...], mask4, b3_ref[...], _OUT_DIM)
    for b in range(_NB):
        blk = lax.slice(o3, (b * _N, 0), ((b + 1) * _N, _HEADS * _OUT_DIM))
        out_ref[b, 0, :] = jnp.mean(blk, axis=0)


def _attn_mat(a):
    heads, outd = a.shape
    return (jnp.eye(heads, dtype=a.dtype)[:, :, None] * a[None, :, :]).reshape(
        heads, heads * outd).T


def kernel(xs, pos_enc, W1, a_src1, a_dst1, b1, W2, a_src2, a_dst2, b2,
           W3, a_src3, a_dst3, b3):
    bs, nr, nc = xs.shape
    enc = pos_enc.shape[-1]
    nblocks = bs * nr
    pe = jnp.broadcast_to(pos_enc[:, None, :, :], (bs, nr, nc, enc))
    x = jnp.concatenate([xs[..., None], pe], axis=-1).reshape(nblocks, nc, 1 + enc)
    mask4 = jnp.repeat(jnp.eye(_HEADS, dtype=jnp.float32), _N, axis=1)

    def whole(shape):
        return pl.BlockSpec(shape, lambda i: tuple(0 for _ in shape))

    hh = _HEADS * _HID
    ho = _HEADS * _OUT_DIM
    out = pl.pallas_call(
        _block_kernel,
        grid=(nblocks // _NB,),
        in_specs=[
            pl.BlockSpec((_NB, nc, 1 + enc), lambda i: (i, 0, 0)),
            whole((_HEADS, _HEADS * _N)),
            whole((1 + enc, hh)),
            whole((hh, _HEADS)), whole((hh, _HEADS)), whole((1, hh)),
            whole((hh, hh)),
            whole((hh, _HEADS)), whole((hh, _HEADS)), whole((1, hh)),
            whole((hh, ho)),
            whole((ho, _HEADS)), whole((ho, _HEADS)), whole((1, ho)),
        ],
        out_specs=pl.BlockSpec((_NB, 1, ho), lambda i: (i, 0, 0)),
        out_shape=jax.ShapeDtypeStruct((nblocks, 1, ho), jnp.float32),
        compiler_params=pltpu.CompilerParams(
            dimension_semantics=("parallel",)),
    )(
        x, mask4,
        W1, _attn_mat(a_src1), _attn_mat(a_dst1), b1.reshape(1, -1),
        W2, _attn_mat(a_src2), _attn_mat(a_dst2), b2.reshape(1, -1),
        W3, _attn_mat(a_src3), _attn_mat(a_dst3), b3.reshape(1, -1),
    )
    return out.reshape(bs, nr, _HEADS * _OUT_DIM)
